# baseline (speedup 1.0000x reference)
import functools

import jax
import jax.numpy as jnp
import numpy as np
from jax import lax
from jax.experimental import pallas as pl
from jax.experimental.pallas import tpu as pltpu
from jax.experimental.pallas import tpu_sc as plsc

_K = 100
_KP = 128
_B = 512
_D = 512
_F_SCORE = 1.2447
_LN_BETA = float(np.log(0.999))
_NC, _NS, _L = 2, 16, 16
_NW = _NC * _NS
_BPW = _B // _NW
_NCH = _D // _L
_ST = 2 * _KP + _L

_mesh = plsc.VectorSubcoreMesh(
    core_axis_name="c", subcore_axis_name="s", num_cores=_NC, num_subcores=_NS)


def _wid():
    return lax.axis_index("c") * _NS + lax.axis_index("s")


def _lane_iota():
    return lax.broadcasted_iota(jnp.int32, (_L,), 0)


def _sqrt16(x):
    xi = lax.bitcast_convert_type(x, jnp.int32)
    yi = jnp.int32(0x5F3759DF) - lax.shift_right_logical(xi, 1)
    y = lax.bitcast_convert_type(yi, jnp.float32)
    for _ in range(4):
        y = y * (1.5 - 0.5 * x * y * y)
    return x * y


def _sdiv(a, b):
    va = jnp.zeros((_L,), jnp.float32) + a
    vb = jnp.zeros((_L,), jnp.float32) + b
    return (va / vb)[0]


def _tc_body(logits_ref, lab_ref, feat_ref, bsum_ref, s_ref):
    x = logits_ref[...]
    labv = lab_ref[...]
    m = jnp.max(x, axis=1, keepdims=True)
    e = jnp.exp(x - m)
    p = e / jnp.sum(e, axis=1, keepdims=True)
    log_p = jnp.maximum(jnp.log(p), -100.0)
    log_1mp = jnp.maximum(jnp.log(1.0 - p), -100.0)
    oh = lax.broadcasted_iota(jnp.int32, x.shape, 1) == labv
    row = (jnp.sum(jnp.where(oh, log_p - log_1mp, 0.0), axis=1, keepdims=True)
           + jnp.sum(log_1mp, axis=1, keepdims=True))
    bsum_ref[...] = -row
    ohp = (lax.broadcasted_iota(jnp.int32, (_B, _KP), 1) == labv
           ).astype(jnp.float32)
    s_ref[...] = lax.dot_general(
        ohp, feat_ref[...], (((0,), (0,)), ((), ())),
        preferred_element_type=jnp.float32,
        precision=lax.Precision.HIGHEST)


def _tc_stage(logits, labels, features):
    return pl.pallas_call(
        _tc_body,
        out_shape=(jax.ShapeDtypeStruct((_B, 1), jnp.float32),
                   jax.ShapeDtypeStruct((_KP, _D), jnp.float32)),
    )(logits, labels.reshape(_B, 1), features)


_SPT = _B // _NS


def _pdc_body(feat_hbm, lab_hbm, s_hbm, cnt_hbm, bsum_hbm,
              stats_out, loss_out,
              feat_v, idx_v, rows, cnt_v, inv_v, stat_v,
              stats_v, lab_v, bsum_v, w_v, loss_v, sem,
              zsum_sm, nz_sm):
    cid = lax.axis_index("c")
    sid = lax.axis_index("s")
    lane = _lane_iota()

    @pl.when(cid == 0)
    def _():
        base = sid * _SPT
        pltpu.sync_copy(lab_hbm.at[pl.ds(base, _SPT)], idx_v.at[pl.ds(0, _SPT)])
        pltpu.sync_copy(cnt_hbm, cnt_v.at[pl.ds(0, _K)])
        gat = pltpu.async_copy(s_hbm.at[idx_v.at[pl.ds(0, _SPT)]], rows, sem)
        pltpu.sync_copy(feat_hbm.at[pl.ds(base, _SPT)], feat_v)
        for h in range(_SPT // _L):
            idxreg = idx_v[pl.ds(h * _L, _L)]
            inv_v[pl.ds(h * _L, _L)] = 1.0 / plsc.load_gather(cnt_v, [idxreg])

        def zb(c, carry):
            zsum_sm[c] = 0.0
            nz_sm[c] = 0.0
            return carry
        lax.fori_loop(0, _KP, zb, 0)
        gat.wait()

        def half(h, sz2):
            hb = h * _L
            idxh = idx_v[pl.ds(hb, _L)]
            invh = inv_v[pl.ds(hb, _L)]
            for i in range(_L):
                inv = invh[i]
                lab = idxh[i]
                acc = jnp.zeros((_L,), jnp.float32)
                for j in range(_NCH):
                    f = feat_v[hb + i, pl.ds(j * _L, _L)]
                    mv = rows[hb + i, pl.ds(j * _L, _L)] * inv
                    acc = acc + jnp.where(f != 0.0, jnp.abs(f - mv), 0.0)
                z = jnp.sum(acc)
                zsum_sm[lab] = zsum_sm[lab] + z
                nz_sm[lab] = nz_sm[lab] + jnp.where(z != 0.0, 1.0, 0.0)
                sz2 = sz2 + z * z
            return sz2
        sz2 = lax.fori_loop(0, _SPT // _L, half, 0.0)

        for q in range(_KP // _L):
            vz = jnp.zeros((_L,), jnp.float32)
            vn = jnp.zeros((_L,), jnp.float32)
            for t in range(_L):
                vz = jnp.where(lane == t, zsum_sm[q * _L + t], vz)
                vn = jnp.where(lane == t, nz_sm[q * _L + t], vn)
            stat_v[pl.ds(q * _L, _L)] = vz
            stat_v[pl.ds(_KP + q * _L, _L)] = vn
        stat_v[pl.ds(2 * _KP, _L)] = jnp.where(lane == 0, sz2, 0.0)
        pltpu.sync_copy(stat_v, stats_out.at[pl.ds(sid * _ST, _ST)])

    plsc.subcore_barrier()

    @pl.when(jnp.logical_and(cid == 0, sid == 0))
    def _():
        pltpu.sync_copy(stats_out, stats_v)
        pltpu.sync_copy(lab_hbm, lab_v)
        pltpu.sync_copy(bsum_hbm, bsum_v)

        sz2_acc = jnp.zeros((_L,), jnp.float32)
        for t in range(_NS):
            sz2_acc = sz2_acc + stats_v[pl.ds(t * _ST + 2 * _KP, _L)]
        sz2 = sz2_acc[0]

        nq = _KP // _L
        zsum_r, nz_r, zim_r, cnt_r, valid_r = [], [], [], [], []
        zm_acc = jnp.zeros((_L,), jnp.float32)
        n_acc = jnp.zeros((_L,), jnp.float32)
        for q in range(nq):
            zsum_c = jnp.zeros((_L,), jnp.float32)
            nz_c = jnp.zeros((_L,), jnp.float32)
            for t in range(_NS):
                zsum_c = zsum_c + stats_v[pl.ds(t * _ST + q * _L, _L)]
                nz_c = nz_c + stats_v[pl.ds(t * _ST + _KP + q * _L, _L)]
            valid = (_lane_iota() + q * _L) < _K
            cnt_c = jnp.where(valid, cnt_v[pl.ds(q * _L, _L)], 1.0)
            zim_c = zsum_c / cnt_c
            zsum_r.append(zsum_c)
            nz_r.append(nz_c)
            zim_r.append(zim_c)
            cnt_r.append(cnt_c)
            valid_r.append(valid)
            zm_acc = zm_acc + jnp.where(valid, zim_c, 0.0)
            n_acc = n_acc + jnp.where(valid, cnt_c, 0.0)
        z_mean = jnp.sum(zm_acc) * (1.0 / _K)
        n_tot = jnp.sum(n_acc)

        cross_acc = jnp.zeros((_L,), jnp.float32)
        for q in range(nq):
            cross_acc = cross_acc + zim_r[q] * (zim_r[q] * nz_r[q]
                                                - 2.0 * zsum_r[q])
        ssw = _sdiv(sz2 + jnp.sum(cross_acc), n_tot - float(_K))

        sb_r = []
        ssb_acc = jnp.zeros((_L,), jnp.float32)
        for q in range(nq):
            dzm = zim_r[q] - z_mean
            sbm = jnp.where(valid_r[q], dzm * dzm * cnt_r[q], 0.0)
            sb_r.append(sbm)
            ssb_acc = ssb_acc + sbm
        ssb = jnp.sum(ssb_acc) * (1.0 / (_K - 1))

        a = z_mean * z_mean
        inv2a = _sdiv(1.0, 2.0 * a)
        ws_acc = jnp.zeros((_L,), jnp.float32)
        for q in range(nq):
            cq = _F_SCORE * ssw * float(_K - 1) - (ssb * float(_K - 1)
                                                   - sb_r[q])
            bq = -(2.0 * z_mean * zsum_r[q] + cq)
            d2 = bq * bq - 4.0 * a * (zsum_r[q] * zsum_r[q])
            dok = d2 >= 0.0
            dq = _sqrt16(jnp.maximum(d2, 0.0))
            n_lb = jnp.abs((-bq - dq) * inv2a)
            n_ub = jnp.abs((-bq + dq) * inv2a)
            c1 = jnp.logical_and(dok, cnt_r[q] < n_lb)
            c2 = jnp.logical_and(dok, cnt_r[q] > n_ub)
            t = jnp.where(c1, 1.0 / (n_lb - cnt_r[q]),
                          jnp.where(c2, 1.0 / (cnt_r[q] - n_ub), 1.0))
            beta = jnp.exp(_LN_BETA * t)
            en = 1.0 - jnp.exp(_LN_BETA * t * cnt_r[q])
            wr = (1.0 - beta) / en
            wrm = jnp.where(valid_r[q], wr, 0.0)
            w_v[pl.ds(q * _L, _L)] = wrm
            ws_acc = ws_acc + wrm
        wsum = jnp.sum(ws_acc)

        dot_acc = jnp.zeros((_L,), jnp.float32)
        for c in range(_B // _L):
            labc = lab_v[pl.ds(c * _L, _L)]
            wg = plsc.load_gather(w_v, [labc])
            dot_acc = dot_acc + wg * bsum_v[pl.ds(c * _L, _L)]
        loss = jnp.sum(dot_acc) * _sdiv(float(_K), wsum) * (1.0 / (_B * _K))
        loss_v[...] = jnp.zeros((_L,), jnp.float32) + loss
        pltpu.sync_copy(loss_v, loss_out)


_phase_dc = functools.partial(
    pl.kernel,
    out_type=(jax.ShapeDtypeStruct((_NS * _ST,), jnp.float32),
              jax.ShapeDtypeStruct((_L,), jnp.float32)),
    mesh=_mesh,
    compiler_params=pltpu.CompilerParams(needs_layout_passes=False),
    scratch_types=[
        pltpu.VMEM((_SPT, _D), jnp.float32),
        pltpu.VMEM((_SPT + _L,), jnp.int32),
        pltpu.VMEM((_SPT, _D), jnp.float32),
        pltpu.VMEM((_KP,), jnp.float32),
        pltpu.VMEM((_SPT + _L,), jnp.float32),
        pltpu.VMEM((_ST,), jnp.float32),
        pltpu.VMEM((_NS * _ST,), jnp.float32),
        pltpu.VMEM((_B,), jnp.int32),
        pltpu.VMEM((_B,), jnp.float32),
        pltpu.VMEM((_KP,), jnp.float32),
        pltpu.VMEM((_L,), jnp.float32),
        pltpu.SemaphoreType.DMA,
        pltpu.SMEM((_KP,), jnp.float32),
        pltpu.SMEM((_KP,), jnp.float32),
    ],
)(_pdc_body)


def kernel(logits, labels, features, sample_num_per_cls):
    labels = labels.astype(jnp.int32)
    bsum, s_tab = _tc_stage(logits, labels, features)
    _, loss_vec = _phase_dc(features, labels, s_tab, sample_num_per_cls,
                            bsum.reshape(_B))
    return loss_vec[0]


# speedup vs baseline: 1.0735x; 1.0735x over previous
import functools

import jax
import jax.numpy as jnp
import numpy as np
from jax import lax
from jax.experimental import pallas as pl
from jax.experimental.pallas import tpu as pltpu
from jax.experimental.pallas import tpu_sc as plsc

_K = 100
_KP = 128
_B = 512
_D = 512
_F_SCORE = 1.2447
_LN_BETA = float(np.log(0.999))
_NC, _NS, _L = 2, 16, 16
_NW = _NC * _NS
_BPW = _B // _NW
_NCH = _D // _L
_ST = 2 * _KP + _L

_mesh = plsc.VectorSubcoreMesh(
    core_axis_name="c", subcore_axis_name="s", num_cores=_NC, num_subcores=_NS)


def _wid():
    return lax.axis_index("c") * _NS + lax.axis_index("s")


def _lane_iota():
    return lax.broadcasted_iota(jnp.int32, (_L,), 0)


def _sqrt16(x):
    xi = lax.bitcast_convert_type(x, jnp.int32)
    yi = jnp.int32(0x5F3759DF) - lax.shift_right_logical(xi, 1)
    y = lax.bitcast_convert_type(yi, jnp.float32)
    for _ in range(4):
        y = y * (1.5 - 0.5 * x * y * y)
    return x * y


def _sdiv(a, b):
    va = jnp.zeros((_L,), jnp.float32) + a
    vb = jnp.zeros((_L,), jnp.float32) + b
    return (va / vb)[0]


def _tc_body(logits_ref, lab_ref, feat_ref, bsum_ref, s_ref):
    x = logits_ref[...]
    labv = lab_ref[...]
    m = jnp.max(x, axis=1, keepdims=True)
    e = jnp.exp(x - m)
    p = e / jnp.sum(e, axis=1, keepdims=True)
    log_p = jnp.maximum(jnp.log(p), -100.0)
    log_1mp = jnp.maximum(jnp.log(1.0 - p), -100.0)
    oh = lax.broadcasted_iota(jnp.int32, x.shape, 1) == labv
    row = (jnp.sum(jnp.where(oh, log_p - log_1mp, 0.0), axis=1, keepdims=True)
           + jnp.sum(log_1mp, axis=1, keepdims=True))
    bsum_ref[...] = -row
    ohp = (lax.broadcasted_iota(jnp.int32, (_B, _KP), 1) == labv
           ).astype(jnp.float32)
    s_ref[...] = lax.dot_general(
        ohp, feat_ref[...], (((0,), (0,)), ((), ())),
        preferred_element_type=jnp.float32,
        precision=lax.Precision.HIGHEST)


def _tc_stage(logits, labels, features):
    return pl.pallas_call(
        _tc_body,
        out_shape=(jax.ShapeDtypeStruct((_B, 1), jnp.float32),
                   jax.ShapeDtypeStruct((_KP, _D), jnp.float32)),
    )(logits, labels.reshape(_B, 1), features)


_SPT = _B // _NS


def _pdc_body(feat_hbm, lab_hbm, s_hbm, cnt_hbm, bsum_hbm,
              stats_out, loss_out,
              feat_v, idx_v, rows, cnt_v, inv_v, stat_v,
              stats_v, lab_v, bsum_v, w_v, loss_v, sem,
              zsum_sm, nz_sm):
    cid = lax.axis_index("c")
    sid = lax.axis_index("s")
    lane = _lane_iota()

    @pl.when(cid == 0)
    def _():
        base = sid * _SPT
        pltpu.sync_copy(lab_hbm.at[pl.ds(base, _SPT)], idx_v.at[pl.ds(0, _SPT)])
        pltpu.sync_copy(cnt_hbm, cnt_v.at[pl.ds(0, _K)])
        gat = pltpu.async_copy(s_hbm.at[idx_v.at[pl.ds(0, _SPT)]], rows, sem)
        pltpu.sync_copy(feat_hbm.at[pl.ds(base, _SPT)], feat_v)
        for h in range(_SPT // _L):
            idxreg = idx_v[pl.ds(h * _L, _L)]
            inv_v[pl.ds(h * _L, _L)] = 1.0 / plsc.load_gather(cnt_v, [idxreg])

        def zb(c, carry):
            zsum_sm[c] = 0.0
            nz_sm[c] = 0.0
            return carry
        lax.fori_loop(0, _KP, zb, 0)
        gat.wait()

        def sample(i, sz2):
            inv = inv_v[pl.ds(i, _L)][0]
            lab = idx_v[pl.ds(i, _L)][0]
            acc = jnp.zeros((_L,), jnp.float32)
            for j in range(_NCH):
                f = feat_v[i, pl.ds(j * _L, _L)]
                mv = rows[i, pl.ds(j * _L, _L)] * inv
                acc = acc + jnp.where(f != 0.0, jnp.abs(f - mv), 0.0)
            z = jnp.sum(acc)
            zsum_sm[lab] = zsum_sm[lab] + z
            nz_sm[lab] = nz_sm[lab] + jnp.where(z != 0.0, 1.0, 0.0)
            return sz2 + z * z
        sz2 = lax.fori_loop(0, _SPT, sample, 0.0)

        for q in range(_KP // _L):
            vz = jnp.zeros((_L,), jnp.float32)
            vn = jnp.zeros((_L,), jnp.float32)
            for t in range(_L):
                vz = jnp.where(lane == t, zsum_sm[q * _L + t], vz)
                vn = jnp.where(lane == t, nz_sm[q * _L + t], vn)
            stat_v[pl.ds(q * _L, _L)] = vz
            stat_v[pl.ds(_KP + q * _L, _L)] = vn
        stat_v[pl.ds(2 * _KP, _L)] = jnp.where(lane == 0, sz2, 0.0)
        pltpu.sync_copy(stat_v, stats_out.at[pl.ds(sid * _ST, _ST)])

    plsc.subcore_barrier()

    @pl.when(jnp.logical_and(cid == 0, sid == 0))
    def _():
        pltpu.sync_copy(stats_out, stats_v)
        pltpu.sync_copy(lab_hbm, lab_v)
        pltpu.sync_copy(bsum_hbm, bsum_v)

        sz2_acc = jnp.zeros((_L,), jnp.float32)
        for t in range(_NS):
            sz2_acc = sz2_acc + stats_v[pl.ds(t * _ST + 2 * _KP, _L)]
        sz2 = sz2_acc[0]

        nq = _KP // _L
        zsum_r, nz_r, zim_r, cnt_r, valid_r = [], [], [], [], []
        zm_acc = jnp.zeros((_L,), jnp.float32)
        n_acc = jnp.zeros((_L,), jnp.float32)
        for q in range(nq):
            zsum_c = jnp.zeros((_L,), jnp.float32)
            nz_c = jnp.zeros((_L,), jnp.float32)
            for t in range(_NS):
                zsum_c = zsum_c + stats_v[pl.ds(t * _ST + q * _L, _L)]
                nz_c = nz_c + stats_v[pl.ds(t * _ST + _KP + q * _L, _L)]
            valid = (_lane_iota() + q * _L) < _K
            cnt_c = jnp.where(valid, cnt_v[pl.ds(q * _L, _L)], 1.0)
            zim_c = zsum_c / cnt_c
            zsum_r.append(zsum_c)
            nz_r.append(nz_c)
            zim_r.append(zim_c)
            cnt_r.append(cnt_c)
            valid_r.append(valid)
            zm_acc = zm_acc + jnp.where(valid, zim_c, 0.0)
            n_acc = n_acc + jnp.where(valid, cnt_c, 0.0)
        z_mean = jnp.sum(zm_acc) * (1.0 / _K)
        n_tot = jnp.sum(n_acc)

        cross_acc = jnp.zeros((_L,), jnp.float32)
        for q in range(nq):
            cross_acc = cross_acc + zim_r[q] * (zim_r[q] * nz_r[q]
                                                - 2.0 * zsum_r[q])
        ssw = _sdiv(sz2 + jnp.sum(cross_acc), n_tot - float(_K))

        sb_r = []
        ssb_acc = jnp.zeros((_L,), jnp.float32)
        for q in range(nq):
            dzm = zim_r[q] - z_mean
            sbm = jnp.where(valid_r[q], dzm * dzm * cnt_r[q], 0.0)
            sb_r.append(sbm)
            ssb_acc = ssb_acc + sbm
        ssb = jnp.sum(ssb_acc) * (1.0 / (_K - 1))

        a = z_mean * z_mean
        inv2a = _sdiv(1.0, 2.0 * a)
        ws_acc = jnp.zeros((_L,), jnp.float32)
        for q in range(nq):
            cq = _F_SCORE * ssw * float(_K - 1) - (ssb * float(_K - 1)
                                                   - sb_r[q])
            bq = -(2.0 * z_mean * zsum_r[q] + cq)
            d2 = bq * bq - 4.0 * a * (zsum_r[q] * zsum_r[q])
            dok = d2 >= 0.0
            dq = _sqrt16(jnp.maximum(d2, 0.0))
            n_lb = jnp.abs((-bq - dq) * inv2a)
            n_ub = jnp.abs((-bq + dq) * inv2a)
            c1 = jnp.logical_and(dok, cnt_r[q] < n_lb)
            c2 = jnp.logical_and(dok, cnt_r[q] > n_ub)
            t = jnp.where(c1, 1.0 / (n_lb - cnt_r[q]),
                          jnp.where(c2, 1.0 / (cnt_r[q] - n_ub), 1.0))
            beta = jnp.exp(_LN_BETA * t)
            en = 1.0 - jnp.exp(_LN_BETA * t * cnt_r[q])
            wr = (1.0 - beta) / en
            wrm = jnp.where(valid_r[q], wr, 0.0)
            w_v[pl.ds(q * _L, _L)] = wrm
            ws_acc = ws_acc + wrm
        wsum = jnp.sum(ws_acc)

        def dotc(c, acc):
            labc = lab_v[pl.ds(c * _L, _L)]
            wg = plsc.load_gather(w_v, [labc])
            return acc + wg * bsum_v[pl.ds(c * _L, _L)]
        dot_acc = lax.fori_loop(0, _B // _L, dotc,
                                jnp.zeros((_L,), jnp.float32))
        loss = jnp.sum(dot_acc) * _sdiv(float(_K), wsum) * (1.0 / (_B * _K))
        loss_v[...] = jnp.zeros((_L,), jnp.float32) + loss
        pltpu.sync_copy(loss_v, loss_out)


_phase_dc = functools.partial(
    pl.kernel,
    out_type=(jax.ShapeDtypeStruct((_NS * _ST,), jnp.float32),
              jax.ShapeDtypeStruct((_L,), jnp.float32)),
    mesh=_mesh,
    compiler_params=pltpu.CompilerParams(needs_layout_passes=False),
    scratch_types=[
        pltpu.VMEM((_SPT, _D), jnp.float32),
        pltpu.VMEM((_SPT + _L,), jnp.int32),
        pltpu.VMEM((_SPT, _D), jnp.float32),
        pltpu.VMEM((_KP,), jnp.float32),
        pltpu.VMEM((_SPT + _L,), jnp.float32),
        pltpu.VMEM((_ST,), jnp.float32),
        pltpu.VMEM((_NS * _ST,), jnp.float32),
        pltpu.VMEM((_B,), jnp.int32),
        pltpu.VMEM((_B,), jnp.float32),
        pltpu.VMEM((_KP,), jnp.float32),
        pltpu.VMEM((_L,), jnp.float32),
        pltpu.SemaphoreType.DMA,
        pltpu.SMEM((_KP,), jnp.float32),
        pltpu.SMEM((_KP,), jnp.float32),
    ],
)(_pdc_body)


def kernel(logits, labels, features, sample_num_per_cls):
    labels = labels.astype(jnp.int32)
    bsum, s_tab = _tc_stage(logits, labels, features)
    _, loss_vec = _phase_dc(features, labels, s_tab, sample_num_per_cls,
                            bsum.reshape(_B))
    return loss_vec[0]
